# baseline (speedup 1.0000x reference)
import functools

import jax
import jax.numpy as jnp
from jax import lax
from jax.experimental import pallas as pl
from jax.experimental.pallas import tpu as pltpu
from jax.experimental.pallas import tpu_sc as plsc

_INFO = plsc.get_sparse_core_info()
_NC = _INFO.num_cores
_NS = _INFO.num_subcores
_NW = _NC * _NS
_CHUNK = 128


def _sc_gather_body(idx_hbm, tab, out, idx_v, rows_v, sem, *, rows_per_w, bpw):
    wid = lax.axis_index("s") * _NC + lax.axis_index("c")
    rbase = wid * rows_per_w
    base = wid * bpw
    pltpu.sync_copy(idx_hbm.at[pl.ds(rbase, rows_per_w)], idx_v)
    handles = []
    for j in range(rows_per_w):
        sl = pl.ds(j * _CHUNK, _CHUNK)
        handles.append(pltpu.async_copy(tab.at[idx_v.at[j]], rows_v.at[sl], sem))
    for h in handles:
        h.wait()
    pltpu.sync_copy(rows_v, out.at[pl.ds(base, bpw)])


@functools.partial(jax.jit, static_argnames=("n", "d"))
def _sc_gather(idx2d, tab, *, n, d):
    bpw = n // _NW
    rows_per_w = bpw // _CHUNK
    mesh = plsc.VectorSubcoreMesh(core_axis_name="c", subcore_axis_name="s")
    f = pl.kernel(
        functools.partial(_sc_gather_body, rows_per_w=rows_per_w, bpw=bpw),
        mesh=mesh,
        out_type=jax.ShapeDtypeStruct((n, d), jnp.float32),
        compiler_params=pltpu.CompilerParams(use_tc_tiling_on_sc=False),
        scratch_types=[
            pltpu.VMEM((rows_per_w, _CHUNK), jnp.int32),
            pltpu.VMEM((bpw, d), jnp.float32),
            pltpu.SemaphoreType.DMA,
        ],
    )
    return f(idx2d, tab)


def _tc_mlp_body(ug, ig, um, im, w1a, w1b, b1, w2, b2, wfa, wfb, bf, out):
    h = um[...] @ w1a[...] + im[...] @ w1b[...] + b1[...]
    h = jnp.maximum(h, 0.0)
    h = jnp.maximum(h @ w2[...] + b2[...], 0.0)
    mf = ug[...] * ig[...]
    out[...] = mf @ wfa[...] + h @ wfb[...] + bf[...]


@functools.partial(jax.jit, static_argnames=("b", "blk"))
def _tc_mlp(rows, w1a, w1b, b1, w2, b2, wfa, wfb, bf, *, b, blk):
    d = rows.shape[1]
    grid = (b // blk,)
    seg = b // blk
    emb = lambda k: pl.BlockSpec((blk, d), lambda i, k=k: (k * seg + i, 0))
    full = lambda a: pl.BlockSpec(a.shape, lambda i: (0,) * a.ndim)
    return pl.pallas_call(
        _tc_mlp_body,
        grid=grid,
        in_specs=[emb(0), emb(1), emb(2), emb(3),
                  full(w1a), full(w1b), full(b1), full(w2), full(b2),
                  full(wfa), full(wfb), full(bf)],
        out_specs=pl.BlockSpec((blk, 1), lambda i: (i, 0)),
        out_shape=jax.ShapeDtypeStruct((b, 1), jnp.float32),
    )(rows, rows, rows, rows, w1a, w1b, b1, w2, b2, wfa, wfb, bf)


def kernel(inputs, user_emb_gmf, item_emb_gmf, user_emb_mlp, item_emb_mlp,
           W1, b1, W2, b2, Wf, bf):
    b = inputs.shape[0]
    v = user_emb_gmf.shape[0]
    d = user_emb_gmf.shape[1]
    tab = jnp.concatenate(
        [user_emb_gmf, item_emb_gmf, user_emb_mlp, item_emb_mlp], axis=0)
    uid = inputs[:, 0]
    iid = inputs[:, 1]
    idx = jnp.concatenate([uid, iid + v, uid + 2 * v, iid + 3 * v])
    idx2d = idx.reshape(-1, _CHUNK)
    rows = _sc_gather(idx2d, tab, n=4 * b, d=d)
    out = _tc_mlp(rows,
                  W1[:d], W1[d:], b1.reshape(1, -1),
                  W2, b2.reshape(1, -1),
                  Wf[:d], Wf[d:], bf.reshape(1, 1),
                  b=b, blk=2048)
    return out


# speedup vs baseline: 1.9111x; 1.9111x over previous
import functools

import jax
import jax.numpy as jnp
from jax import lax
from jax.experimental import pallas as pl
from jax.experimental.pallas import tpu as pltpu
from jax.experimental.pallas import tpu_sc as plsc

_INFO = plsc.get_sparse_core_info()
_NC = _INFO.num_cores
_NS = _INFO.num_subcores
_NW = _NC * _NS
_CHUNK = 128


def _sc_gather_body(uidx_hbm, iidx_hbm, upack, ipack, uout, iout,
                    idx_v, rows_v, sem, *, rows_per_w, bpw):
    wid = lax.axis_index("s") * _NC + lax.axis_index("c")
    rbase = wid * rows_per_w
    base = wid * bpw
    for (src_idx, tab, out) in ((uidx_hbm, upack, uout),
                                (iidx_hbm, ipack, iout)):
        pltpu.sync_copy(src_idx.at[pl.ds(rbase, rows_per_w)], idx_v)
        handles = []
        for j in range(rows_per_w):
            sl = pl.ds(j * _CHUNK, _CHUNK)
            handles.append(
                pltpu.async_copy(tab.at[idx_v.at[j]], rows_v.at[sl], sem))
        for h in handles:
            h.wait()
        pltpu.sync_copy(rows_v, out.at[pl.ds(base, bpw)])


@functools.partial(jax.jit, static_argnames=("b",))
def _sc_gather(uidx2d, iidx2d, upack, ipack, *, b):
    bpw = b // _NW
    rows_per_w = bpw // _CHUNK
    w = upack.shape[1]
    mesh = plsc.VectorSubcoreMesh(core_axis_name="c", subcore_axis_name="s")
    out_sh = jax.ShapeDtypeStruct((b, w), jnp.float32)
    f = pl.kernel(
        functools.partial(_sc_gather_body, rows_per_w=rows_per_w, bpw=bpw),
        mesh=mesh,
        out_type=(out_sh, out_sh),
        scratch_types=[
            pltpu.VMEM((rows_per_w, _CHUNK), jnp.int32),
            pltpu.VMEM((bpw, w), jnp.float32),
            pltpu.SemaphoreType.DMA,
        ],
    )
    return f(uidx2d, iidx2d, upack, ipack)


def _tc_mlp_body(urows, irows, upar, ipar, w1a, w1b, b1, w2, b2,
                 wfa, wfb, bf, out):
    d = w1a.shape[0]
    usel = jnp.where(upar[...] > 0, urows[:, 2 * d:], urows[:, :2 * d])
    isel = jnp.where(ipar[...] > 0, irows[:, 2 * d:], irows[:, :2 * d])
    ug, um = usel[:, :d], usel[:, d:]
    ig, im = isel[:, :d], isel[:, d:]
    h = um @ w1a[...] + im @ w1b[...] + b1[...]
    h = jnp.maximum(h, 0.0)
    h = jnp.maximum(h @ w2[...] + b2[...], 0.0)
    out[...] = (ug * ig) @ wfa[...] + h @ wfb[...] + bf[...]


@functools.partial(jax.jit, static_argnames=("blk",))
def _tc_mlp(urows, irows, upar, ipar, w1a, w1b, b1, w2, b2, wfa, wfb, bf,
            *, blk):
    b, w = urows.shape
    grid = (b // blk,)
    row_spec = pl.BlockSpec((blk, w), lambda i: (i, 0))
    par_spec = pl.BlockSpec((blk, 1), lambda i: (i, 0))
    full = lambda a: pl.BlockSpec(a.shape, lambda i: (0,) * a.ndim)
    return pl.pallas_call(
        _tc_mlp_body,
        grid=grid,
        in_specs=[row_spec, row_spec, par_spec, par_spec,
                  full(w1a), full(w1b), full(b1), full(w2), full(b2),
                  full(wfa), full(wfb), full(bf)],
        out_specs=pl.BlockSpec((blk, 1), lambda i: (i, 0)),
        out_shape=jax.ShapeDtypeStruct((b, 1), jnp.float32),
    )(urows, irows, upar, ipar, w1a, w1b, b1, w2, b2, wfa, wfb, bf)


def kernel(inputs, user_emb_gmf, item_emb_gmf, user_emb_mlp, item_emb_mlp,
           W1, b1, W2, b2, Wf, bf):
    b = inputs.shape[0]
    d = user_emb_gmf.shape[1]
    uid = inputs[:, 0]
    iid = inputs[:, 1]
    upack = jnp.concatenate([user_emb_gmf, user_emb_mlp],
                            axis=1).reshape(-1, 4 * d)
    ipack = jnp.concatenate([item_emb_gmf, item_emb_mlp],
                            axis=1).reshape(-1, 4 * d)
    uidx2d = (uid // 2).reshape(-1, _CHUNK)
    iidx2d = (iid // 2).reshape(-1, _CHUNK)
    urows, irows = _sc_gather(uidx2d, iidx2d, upack, ipack, b=b)
    out = _tc_mlp(urows, irows,
                  (uid % 2).reshape(-1, 1), (iid % 2).reshape(-1, 1),
                  W1[:d], W1[d:], b1.reshape(1, -1),
                  W2, b2.reshape(1, -1),
                  Wf[:d], Wf[d:], bf.reshape(1, 1),
                  blk=2048)
    return out
